# baseline (speedup 1.0000x reference)
import functools

import jax
import jax.numpy as jnp
from jax import lax
from jax.experimental import pallas as pl
from jax.experimental.pallas import tpu as pltpu
from jax.experimental.pallas import tpu_sc as plsc

N = 10000
E = 320000
C = 128
K = 16
WF = 4 * C

NC = 2
NS = 16
NW = NC * NS
CH = 80
NCHUNK0 = 62
NCHUNK1 = 63
EPW0 = NCHUNK0 * CH
EPW1 = NCHUNK1 * CH
EH0 = EPW0 * NW
EH1 = EPW1 * NW
N2 = 10240
RPT = N2 // NS
ZB = 16

_LANES = C // 16


def _edge_kernel(kernel_basis, Wk, row_off, rows):
    eb = 1280
    off_blk = row_off // eb

    def body(a_ref, w_ref, o_ref):
        o_ref[...] = jnp.dot(a_ref[...], w_ref[...],
                             preferred_element_type=jnp.float32)

    return pl.pallas_call(
        body,
        grid=(rows // eb,),
        in_specs=[
            pl.BlockSpec((eb, K), lambda i: (i + off_blk, 0)),
            pl.BlockSpec((K, C), lambda i: (0, 0)),
        ],
        out_specs=pl.BlockSpec((eb, C), lambda i: (i, 0)),
        out_shape=jax.ShapeDtypeStruct((rows, C), jnp.float32),
    )(kernel_basis, Wk.T)


def _sc_segment(x, kern_half, eidx_flat, ebase, epw, nchunk):
    mesh = plsc.VectorSubcoreMesh(core_axis_name="c", subcore_axis_name="s")

    @functools.partial(
        pl.kernel,
        out_type=jax.ShapeDtypeStruct((NC, N2, C), jnp.float32),
        mesh=mesh,
        scratch_types=[
            pltpu.VMEM((4, CH), jnp.int32),
            pltpu.VMEM((4, CH), jnp.int32),
            pltpu.VMEM((2, CH, C), jnp.float32),
            pltpu.VMEM((2, CH, C), jnp.float32),
            pltpu.VMEM((ZB, C), jnp.float32),
            pltpu.VMEM_SHARED((N2, C), jnp.float32),
            pltpu.SemaphoreType.DMA((4,)),
            pltpu.SemaphoreType.DMA((2,)),
            pltpu.SemaphoreType.DMA((2,)),
        ],
    )
    def sc(x_hbm, kern_hbm, eidx_hbm, out_hbm,
           src_v, dst_v, xs_v, kern_v, zero_v, acc_sh,
           sem_i, sem_k, sem_g):
        cid = lax.axis_index("c")
        sid = lax.axis_index("s")
        wid = sid * NC + cid
        e0 = wid * epw
        s0 = ebase + e0
        d0 = E + ebase + e0

        z16 = jnp.zeros((16,), jnp.float32)
        for i in range(ZB):
            for c in range(_LANES):
                zero_v[i, pl.ds(c * 16, 16)] = z16
        for r in range(RPT // ZB):
            pltpu.sync_copy(zero_v, acc_sh.at[pl.ds(sid * RPT + r * ZB, ZB)])
        plsc.subcore_barrier()

        def issue_idx(t, ib):
            pltpu.async_copy(eidx_hbm.at[pl.ds(s0 + t * CH, CH)],
                             src_v.at[ib], sem_i.at[ib])
            pltpu.async_copy(eidx_hbm.at[pl.ds(d0 + t * CH, CH)],
                             dst_v.at[ib], sem_i.at[ib])

        def wait_idx(t, ib):
            pltpu.make_async_copy(eidx_hbm.at[pl.ds(s0 + t * CH, CH)],
                                  src_v.at[ib], sem_i.at[ib]).wait()
            pltpu.make_async_copy(eidx_hbm.at[pl.ds(d0 + t * CH, CH)],
                                  dst_v.at[ib], sem_i.at[ib]).wait()

        def issue_kern(t, b):
            pltpu.async_copy(kern_hbm.at[pl.ds(e0 + t * CH, CH)],
                             kern_v.at[b], sem_k.at[b])

        def wait_kern(t, b):
            pltpu.make_async_copy(kern_hbm.at[pl.ds(e0 + t * CH, CH)],
                                  kern_v.at[b], sem_k.at[b]).wait()

        def issue_gather(b, ib):
            pltpu.async_copy(x_hbm.at[src_v.at[ib]], xs_v.at[b], sem_g.at[b])

        def wait_gather(b, ib):
            pltpu.make_async_copy(x_hbm.at[src_v.at[ib]], xs_v.at[b],
                                  sem_g.at[b]).wait()

        for t in (0, 1):
            issue_idx(t, t)
            issue_kern(t, t)
        wait_idx(0, 0)
        issue_gather(0, 0)

        def chunk(j, carry):
            b = lax.rem(j, 2)
            ib = lax.rem(j, 4)
            bn = lax.rem(j + 1, 2)
            ibn = lax.rem(j + 1, 4)

            @pl.when(j + 2 < nchunk)
            def _():
                issue_idx(j + 2, lax.rem(j + 2, 4))

            @pl.when(j + 1 < nchunk)
            def _():
                wait_idx(j + 1, ibn)
                issue_gather(bn, ibn)

            wait_kern(j, b)
            wait_gather(b, ib)

            @plsc.parallel_loop(0, CH, 1, unroll=4)
            def erow(e):
                for c in range(_LANES):
                    sl = pl.ds(c * 16, 16)
                    xs_v[b, e, sl] = xs_v[b, e, sl] * kern_v[b, e, sl]

            @pl.when(j + 2 < nchunk)
            def _():
                issue_kern(j + 2, b)

            pltpu.sync_copy(xs_v.at[b], acc_sh.at[dst_v.at[ib]], add=True)
            return carry

        lax.fori_loop(0, nchunk, chunk, 0)
        plsc.subcore_barrier()

        for r in range(RPT // ZB):
            off = sid * RPT + r * ZB
            pltpu.sync_copy(acc_sh.at[pl.ds(off, ZB)],
                            out_hbm.at[cid, pl.ds(off, ZB)])

    return sc(x, kern_half, eidx_flat)


def _ln_mlp(p0, p1, x, conv_bias, ln_g, ln_b, W1T, b1, W2T, b2, layer_scale):
    nb = 1000

    def body(p0_ref, p1_ref, x_ref, cb_ref, g_ref, b_ref, w1_ref, b1_ref,
             w2_ref, b2_ref, ls_ref, o_ref):
        x1 = (p0_ref[0] + p0_ref[1]) + (p1_ref[0] + p1_ref[1]) + cb_ref[...]
        mu = jnp.mean(x1, axis=-1, keepdims=True)
        xc = x1 - mu
        var = jnp.mean(xc * xc, axis=-1, keepdims=True)
        h = xc * lax.rsqrt(var + 1e-5) * g_ref[...] + b_ref[...]
        a = jnp.dot(h, w1_ref[...], preferred_element_type=jnp.float32)
        a = a + b1_ref[...]
        a = 0.5 * a * (1.0 + lax.erf(a * 0.7071067811865476))
        o = jnp.dot(a, w2_ref[...], preferred_element_type=jnp.float32)
        o = o + b2_ref[...]
        o_ref[...] = ls_ref[...] * o + x_ref[...]

    part_spec = pl.BlockSpec((NC, nb, C), lambda i: (0, i, 0))
    return pl.pallas_call(
        body,
        grid=(N // nb,),
        in_specs=[
            part_spec,
            part_spec,
            pl.BlockSpec((nb, C), lambda i: (i, 0)),
            pl.BlockSpec((1, C), lambda i: (0, 0)),
            pl.BlockSpec((1, C), lambda i: (0, 0)),
            pl.BlockSpec((1, C), lambda i: (0, 0)),
            pl.BlockSpec((C, WF), lambda i: (0, 0)),
            pl.BlockSpec((1, WF), lambda i: (0, 0)),
            pl.BlockSpec((WF, C), lambda i: (0, 0)),
            pl.BlockSpec((1, C), lambda i: (0, 0)),
            pl.BlockSpec((1, C), lambda i: (0, 0)),
        ],
        out_specs=pl.BlockSpec((nb, C), lambda i: (i, 0)),
        out_shape=jax.ShapeDtypeStruct((N, C), jnp.float32),
    )(p0, p1, x, conv_bias.reshape(1, C), ln_g.reshape(1, C),
      ln_b.reshape(1, C), W1T, b1.reshape(1, WF), W2T, b2.reshape(1, C),
      layer_scale.reshape(1, C))


def kernel(x, kernel_basis, fiber_kernel_basis, edge_index, Wk, conv_bias,
           ln_g, ln_b, W1, b1, W2, b2, layer_scale):
    eidx_flat = edge_index.reshape(2 * E)
    kern0 = _edge_kernel(kernel_basis, Wk, 0, EH0)
    kern1 = _edge_kernel(kernel_basis, Wk, EH0, EH1)
    p0 = _sc_segment(x, kern0, eidx_flat, 0, EPW0, NCHUNK0)
    p1 = _sc_segment(x, kern1, eidx_flat, EH0, EPW1, NCHUNK1)
    return _ln_mlp(p0, p1, x, conv_bias, ln_g, ln_b,
                   W1.T, b1, W2.T, b2, layer_scale)


# speedup vs baseline: 1.1354x; 1.1354x over previous
import functools

import jax
import jax.numpy as jnp
from jax import lax
from jax.experimental import pallas as pl
from jax.experimental.pallas import tpu as pltpu
from jax.experimental.pallas import tpu_sc as plsc

N = 10000
E = 320000
C = 128
K = 16
WF = 4 * C

NC = 2
NS = 16
NW = NC * NS
CH = 80
EPW = E // NW
NCHUNK = EPW // CH
N2 = 10240
RPT = N2 // NS
ZB = 16

_LANES = C // 16


def _edge_kernel(kernel_basis, Wk, row_off, rows):
    eb = 8000
    off_blk = row_off // eb

    def body(a_ref, w_ref, o_ref):
        o_ref[...] = jnp.dot(a_ref[...], w_ref[...],
                             preferred_element_type=jnp.float32)

    return pl.pallas_call(
        body,
        grid=(rows // eb,),
        in_specs=[
            pl.BlockSpec((eb, K), lambda i: (i + off_blk, 0)),
            pl.BlockSpec((K, C), lambda i: (0, 0)),
        ],
        out_specs=pl.BlockSpec((eb, C), lambda i: (i, 0)),
        out_shape=jax.ShapeDtypeStruct((rows, C), jnp.float32),
    )(kernel_basis, Wk.T)


def _sc_segment(x, kern_half, eidx_flat, ebase, epw, nchunk):
    mesh = plsc.VectorSubcoreMesh(core_axis_name="c", subcore_axis_name="s")

    @functools.partial(
        pl.kernel,
        out_type=jax.ShapeDtypeStruct((NC, N2, C), jnp.float32),
        mesh=mesh,
        scratch_types=[
            pltpu.VMEM((4, CH), jnp.int32),
            pltpu.VMEM((4, CH), jnp.int32),
            pltpu.VMEM((2, CH, C), jnp.float32),
            pltpu.VMEM((2, CH, C), jnp.float32),
            pltpu.VMEM((ZB, C), jnp.float32),
            pltpu.VMEM_SHARED((N2, C), jnp.float32),
            pltpu.SemaphoreType.DMA((4,)),
            pltpu.SemaphoreType.DMA((2,)),
            pltpu.SemaphoreType.DMA((2,)),
        ],
    )
    def sc(x_hbm, kern_hbm, eidx_hbm, out_hbm,
           src_v, dst_v, xs_v, kern_v, zero_v, acc_sh,
           sem_i, sem_k, sem_g):
        cid = lax.axis_index("c")
        sid = lax.axis_index("s")
        wid = sid * NC + cid
        e0 = wid * epw
        s0 = ebase + e0
        d0 = E + ebase + e0

        z16 = jnp.zeros((16,), jnp.float32)
        for i in range(ZB):
            for c in range(_LANES):
                zero_v[i, pl.ds(c * 16, 16)] = z16
        for r in range(RPT // ZB):
            pltpu.sync_copy(zero_v, acc_sh.at[pl.ds(sid * RPT + r * ZB, ZB)])
        plsc.subcore_barrier()

        def issue_idx(t, ib):
            pltpu.async_copy(eidx_hbm.at[pl.ds(s0 + t * CH, CH)],
                             src_v.at[ib], sem_i.at[ib])
            pltpu.async_copy(eidx_hbm.at[pl.ds(d0 + t * CH, CH)],
                             dst_v.at[ib], sem_i.at[ib])

        def wait_idx(t, ib):
            pltpu.make_async_copy(eidx_hbm.at[pl.ds(s0 + t * CH, CH)],
                                  src_v.at[ib], sem_i.at[ib]).wait()
            pltpu.make_async_copy(eidx_hbm.at[pl.ds(d0 + t * CH, CH)],
                                  dst_v.at[ib], sem_i.at[ib]).wait()

        def issue_kern(t, b):
            pltpu.async_copy(kern_hbm.at[pl.ds(e0 + t * CH, CH)],
                             kern_v.at[b], sem_k.at[b])

        def wait_kern(t, b):
            pltpu.make_async_copy(kern_hbm.at[pl.ds(e0 + t * CH, CH)],
                                  kern_v.at[b], sem_k.at[b]).wait()

        def issue_gather(b, ib):
            pltpu.async_copy(x_hbm.at[src_v.at[ib]], xs_v.at[b], sem_g.at[b])

        def wait_gather(b, ib):
            pltpu.make_async_copy(x_hbm.at[src_v.at[ib]], xs_v.at[b],
                                  sem_g.at[b]).wait()

        for t in (0, 1):
            issue_idx(t, t)
            issue_kern(t, t)
        wait_idx(0, 0)
        issue_gather(0, 0)

        def chunk(j, carry):
            b = lax.rem(j, 2)
            ib = lax.rem(j, 4)
            bn = lax.rem(j + 1, 2)
            ibn = lax.rem(j + 1, 4)

            @pl.when(j + 2 < nchunk)
            def _():
                issue_idx(j + 2, lax.rem(j + 2, 4))

            @pl.when(j + 1 < nchunk)
            def _():
                wait_idx(j + 1, ibn)
                issue_gather(bn, ibn)

            wait_kern(j, b)
            wait_gather(b, ib)

            @plsc.parallel_loop(0, CH, 1, unroll=4)
            def erow(e):
                for c in range(_LANES):
                    sl = pl.ds(c * 16, 16)
                    xs_v[b, e, sl] = xs_v[b, e, sl] * kern_v[b, e, sl]

            @pl.when(j + 2 < nchunk)
            def _():
                issue_kern(j + 2, b)

            pltpu.sync_copy(xs_v.at[b], acc_sh.at[dst_v.at[ib]], add=True)
            return carry

        lax.fori_loop(0, nchunk, chunk, 0)
        plsc.subcore_barrier()

        for r in range(RPT // ZB):
            off = sid * RPT + r * ZB
            pltpu.sync_copy(acc_sh.at[pl.ds(off, ZB)],
                            out_hbm.at[cid, pl.ds(off, ZB)])

    return sc(x, kern_half, eidx_flat)


def _ln_mlp(partials, x, conv_bias, ln_g, ln_b, W1T, b1, W2T, b2, layer_scale):
    nb = 1000

    def body(p_ref, x_ref, cb_ref, g_ref, b_ref, w1_ref, b1_ref,
             w2_ref, b2_ref, ls_ref, o_ref):
        x1 = p_ref[0] + p_ref[1] + cb_ref[...]
        mu = jnp.mean(x1, axis=-1, keepdims=True)
        xc = x1 - mu
        var = jnp.mean(xc * xc, axis=-1, keepdims=True)
        h = xc * lax.rsqrt(var + 1e-5) * g_ref[...] + b_ref[...]
        a = jnp.dot(h, w1_ref[...], preferred_element_type=jnp.float32)
        a = a + b1_ref[...]
        a = 0.5 * a * (1.0 + lax.erf(a * 0.7071067811865476))
        o = jnp.dot(a, w2_ref[...], preferred_element_type=jnp.float32)
        o = o + b2_ref[...]
        o_ref[...] = ls_ref[...] * o + x_ref[...]

    part_spec = pl.BlockSpec((NC, nb, C), lambda i: (0, i, 0))
    return pl.pallas_call(
        body,
        grid=(N // nb,),
        in_specs=[
            part_spec,
            pl.BlockSpec((nb, C), lambda i: (i, 0)),
            pl.BlockSpec((1, C), lambda i: (0, 0)),
            pl.BlockSpec((1, C), lambda i: (0, 0)),
            pl.BlockSpec((1, C), lambda i: (0, 0)),
            pl.BlockSpec((C, WF), lambda i: (0, 0)),
            pl.BlockSpec((1, WF), lambda i: (0, 0)),
            pl.BlockSpec((WF, C), lambda i: (0, 0)),
            pl.BlockSpec((1, C), lambda i: (0, 0)),
            pl.BlockSpec((1, C), lambda i: (0, 0)),
        ],
        out_specs=pl.BlockSpec((nb, C), lambda i: (i, 0)),
        out_shape=jax.ShapeDtypeStruct((N, C), jnp.float32),
    )(partials, x, conv_bias.reshape(1, C), ln_g.reshape(1, C),
      ln_b.reshape(1, C), W1T, b1.reshape(1, WF), W2T, b2.reshape(1, C),
      layer_scale.reshape(1, C))


def kernel(x, kernel_basis, fiber_kernel_basis, edge_index, Wk, conv_bias,
           ln_g, ln_b, W1, b1, W2, b2, layer_scale):
    eidx_flat = edge_index.reshape(2 * E)
    kern_full = _edge_kernel(kernel_basis, Wk, 0, E)
    partials = _sc_segment(x, kern_full, eidx_flat, 0, EPW, NCHUNK)
    return _ln_mlp(partials, x, conv_bias, ln_g, ln_b,
                   W1.T, b1, W2.T, b2, layer_scale)


# speedup vs baseline: 1.1371x; 1.0015x over previous
import functools

import jax
import jax.numpy as jnp
from jax import lax
from jax.experimental import pallas as pl
from jax.experimental.pallas import tpu as pltpu
from jax.experimental.pallas import tpu_sc as plsc

N = 10000
E = 320000
C = 128
K = 16
WF = 4 * C

NC = 2
NS = 16
NW = NC * NS
CH = 80
EPW = E // NW
NCHUNK = EPW // CH
N2 = 10240
RPT = N2 // NS
ZB = 16

_LANES = C // 16


def _edge_kernel(kernel_basis, Wk, row_off, rows):
    eb = 8000
    off_blk = row_off // eb

    def body(a_ref, w_ref, o_ref):
        o_ref[...] = jnp.dot(a_ref[...], w_ref[...],
                             preferred_element_type=jnp.float32)

    return pl.pallas_call(
        body,
        grid=(rows // eb,),
        in_specs=[
            pl.BlockSpec((eb, K), lambda i: (i + off_blk, 0)),
            pl.BlockSpec((K, C), lambda i: (0, 0)),
        ],
        out_specs=pl.BlockSpec((eb, C), lambda i: (i, 0)),
        out_shape=jax.ShapeDtypeStruct((rows, C), jnp.float32),
    )(kernel_basis, Wk.T)


def _sc_segment(x, kern_half, eidx_flat, ebase, epw, nchunk):
    mesh = plsc.VectorSubcoreMesh(core_axis_name="c", subcore_axis_name="s")

    @functools.partial(
        pl.kernel,
        out_type=jax.ShapeDtypeStruct((NC, N2, C), jnp.float32),
        mesh=mesh,
        scratch_types=[
            pltpu.VMEM((4, CH), jnp.int32),
            pltpu.VMEM((4, CH), jnp.int32),
            pltpu.VMEM((2, CH, C), jnp.float32),
            pltpu.VMEM((2, CH, C), jnp.float32),
            pltpu.VMEM((ZB, C), jnp.float32),
            pltpu.VMEM_SHARED((N2, C), jnp.float32),
            pltpu.SemaphoreType.DMA((4,)),
            pltpu.SemaphoreType.DMA((2,)),
            pltpu.SemaphoreType.DMA((2,)),
            pltpu.SemaphoreType.DMA((2,)),
        ],
    )
    def sc(x_hbm, kern_hbm, eidx_hbm, out_hbm,
           src_v, dst_v, xs_v, kern_v, zero_v, acc_sh,
           sem_i, sem_k, sem_g, sem_s):
        cid = lax.axis_index("c")
        sid = lax.axis_index("s")
        wid = sid * NC + cid
        e0 = wid * epw
        s0 = ebase + e0
        d0 = E + ebase + e0

        z16 = jnp.zeros((16,), jnp.float32)
        for i in range(ZB):
            for c in range(_LANES):
                zero_v[i, pl.ds(c * 16, 16)] = z16
        for r in range(RPT // ZB):
            pltpu.sync_copy(zero_v, acc_sh.at[pl.ds(sid * RPT + r * ZB, ZB)])
        plsc.subcore_barrier()

        def issue_idx(t, ib):
            pltpu.async_copy(eidx_hbm.at[pl.ds(s0 + t * CH, CH)],
                             src_v.at[ib], sem_i.at[ib])
            pltpu.async_copy(eidx_hbm.at[pl.ds(d0 + t * CH, CH)],
                             dst_v.at[ib], sem_i.at[ib])

        def wait_idx(t, ib):
            pltpu.make_async_copy(eidx_hbm.at[pl.ds(s0 + t * CH, CH)],
                                  src_v.at[ib], sem_i.at[ib]).wait()
            pltpu.make_async_copy(eidx_hbm.at[pl.ds(d0 + t * CH, CH)],
                                  dst_v.at[ib], sem_i.at[ib]).wait()

        def issue_kern(t, b):
            pltpu.async_copy(kern_hbm.at[pl.ds(e0 + t * CH, CH)],
                             kern_v.at[b], sem_k.at[b])

        def wait_kern(t, b):
            pltpu.make_async_copy(kern_hbm.at[pl.ds(e0 + t * CH, CH)],
                                  kern_v.at[b], sem_k.at[b]).wait()

        def issue_gather(b, ib):
            pltpu.async_copy(x_hbm.at[src_v.at[ib]], xs_v.at[b], sem_g.at[b])

        def wait_gather(b, ib):
            pltpu.make_async_copy(x_hbm.at[src_v.at[ib]], xs_v.at[b],
                                  sem_g.at[b]).wait()

        def issue_scatter(b, ib):
            pltpu.async_copy(xs_v.at[b], acc_sh.at[dst_v.at[ib]],
                             sem_s.at[b], add=True)

        def wait_scatter(b, ib):
            pltpu.make_async_copy(xs_v.at[b], acc_sh.at[dst_v.at[ib]],
                                  sem_s.at[b]).wait()

        for t in (0, 1):
            issue_idx(t, t)
            issue_kern(t, t)
        wait_idx(0, 0)
        issue_gather(0, 0)

        def chunk(j, carry):
            b = lax.rem(j, 2)
            ib = lax.rem(j, 4)
            bn = lax.rem(j + 1, 2)
            ibn = lax.rem(j + 1, 4)

            @pl.when(j + 2 < nchunk)
            def _():
                issue_idx(j + 2, lax.rem(j + 2, 4))

            @pl.when(j + 1 < nchunk)
            def _():
                wait_idx(j + 1, ibn)

                @pl.when(j >= 1)
                def _():
                    wait_scatter(bn, lax.rem(j + 3, 4))

                issue_gather(bn, ibn)

            wait_kern(j, b)
            wait_gather(b, ib)

            @plsc.parallel_loop(0, CH, 1, unroll=4)
            def erow(e):
                for c in range(_LANES):
                    sl = pl.ds(c * 16, 16)
                    xs_v[b, e, sl] = xs_v[b, e, sl] * kern_v[b, e, sl]

            @pl.when(j + 2 < nchunk)
            def _():
                issue_kern(j + 2, b)

            issue_scatter(b, ib)
            return carry

        lax.fori_loop(0, nchunk, chunk, 0)
        for t in (nchunk - 2, nchunk - 1):
            wait_scatter(t % 2, t % 4)
        plsc.subcore_barrier()

        for r in range(RPT // ZB):
            off = sid * RPT + r * ZB
            pltpu.sync_copy(acc_sh.at[pl.ds(off, ZB)],
                            out_hbm.at[cid, pl.ds(off, ZB)])

    return sc(x, kern_half, eidx_flat)


def _ln_mlp(partials, x, conv_bias, ln_g, ln_b, W1T, b1, W2T, b2, layer_scale):
    nb = 1000

    def body(p_ref, x_ref, cb_ref, g_ref, b_ref, w1_ref, b1_ref,
             w2_ref, b2_ref, ls_ref, o_ref):
        x1 = p_ref[0] + p_ref[1] + cb_ref[...]
        mu = jnp.mean(x1, axis=-1, keepdims=True)
        xc = x1 - mu
        var = jnp.mean(xc * xc, axis=-1, keepdims=True)
        h = xc * lax.rsqrt(var + 1e-5) * g_ref[...] + b_ref[...]
        a = jnp.dot(h, w1_ref[...], preferred_element_type=jnp.float32)
        a = a + b1_ref[...]
        a = 0.5 * a * (1.0 + lax.erf(a * 0.7071067811865476))
        o = jnp.dot(a, w2_ref[...], preferred_element_type=jnp.float32)
        o = o + b2_ref[...]
        o_ref[...] = ls_ref[...] * o + x_ref[...]

    part_spec = pl.BlockSpec((NC, nb, C), lambda i: (0, i, 0))
    return pl.pallas_call(
        body,
        grid=(N // nb,),
        in_specs=[
            part_spec,
            pl.BlockSpec((nb, C), lambda i: (i, 0)),
            pl.BlockSpec((1, C), lambda i: (0, 0)),
            pl.BlockSpec((1, C), lambda i: (0, 0)),
            pl.BlockSpec((1, C), lambda i: (0, 0)),
            pl.BlockSpec((C, WF), lambda i: (0, 0)),
            pl.BlockSpec((1, WF), lambda i: (0, 0)),
            pl.BlockSpec((WF, C), lambda i: (0, 0)),
            pl.BlockSpec((1, C), lambda i: (0, 0)),
            pl.BlockSpec((1, C), lambda i: (0, 0)),
        ],
        out_specs=pl.BlockSpec((nb, C), lambda i: (i, 0)),
        out_shape=jax.ShapeDtypeStruct((N, C), jnp.float32),
    )(partials, x, conv_bias.reshape(1, C), ln_g.reshape(1, C),
      ln_b.reshape(1, C), W1T, b1.reshape(1, WF), W2T, b2.reshape(1, C),
      layer_scale.reshape(1, C))


def kernel(x, kernel_basis, fiber_kernel_basis, edge_index, Wk, conv_bias,
           ln_g, ln_b, W1, b1, W2, b2, layer_scale):
    eidx_flat = edge_index.reshape(2 * E)
    kern_full = _edge_kernel(kernel_basis, Wk, 0, E)
    partials = _sc_segment(x, kern_full, eidx_flat, 0, EPW, NCHUNK)
    return _ln_mlp(partials, x, conv_bias, ln_g, ln_b,
                   W1.T, b1, W2.T, b2, layer_scale)


# speedup vs baseline: 1.1501x; 1.0114x over previous
import functools

import jax
import jax.numpy as jnp
from jax import lax
from jax.experimental import pallas as pl
from jax.experimental.pallas import tpu as pltpu
from jax.experimental.pallas import tpu_sc as plsc

N = 10000
E = 320000
C = 128
K = 16
WF = 4 * C

NC = 2
NS = 16
NW = NC * NS
CH = 80
EPW = E // NW
NCHUNK = EPW // CH
N2 = 10240
RPT = N2 // NS
ZB = 16

_LANES = C // 16


def _edge_kernel(kernel_basis, Wk, row_off, rows):
    eb = 16000
    off_blk = row_off // eb

    def body(a_ref, w_ref, o_ref):
        o_ref[...] = jnp.dot(a_ref[...], w_ref[...],
                             preferred_element_type=jnp.float32)

    return pl.pallas_call(
        body,
        grid=(rows // eb,),
        in_specs=[
            pl.BlockSpec((eb, K), lambda i: (i + off_blk, 0)),
            pl.BlockSpec((K, C), lambda i: (0, 0)),
        ],
        out_specs=pl.BlockSpec((eb, C), lambda i: (i, 0)),
        out_shape=jax.ShapeDtypeStruct((rows, C), jnp.float32),
    )(kernel_basis, Wk.T)


def _sc_segment(x, kern_half, eidx_flat, ebase, epw, nchunk):
    mesh = plsc.VectorSubcoreMesh(core_axis_name="c", subcore_axis_name="s")

    @functools.partial(
        pl.kernel,
        out_type=jax.ShapeDtypeStruct((NC, N2, C), jnp.float32),
        mesh=mesh,
        scratch_types=[
            pltpu.VMEM((4, CH), jnp.int32),
            pltpu.VMEM((4, CH), jnp.int32),
            pltpu.VMEM((2, CH, C), jnp.float32),
            pltpu.VMEM((2, CH, C), jnp.float32),
            pltpu.VMEM((ZB, C), jnp.float32),
            pltpu.VMEM_SHARED((N2, C), jnp.float32),
            pltpu.SemaphoreType.DMA((4,)),
            pltpu.SemaphoreType.DMA((2,)),
            pltpu.SemaphoreType.DMA((2,)),
            pltpu.SemaphoreType.DMA((2,)),
        ],
    )
    def sc(x_hbm, kern_hbm, eidx_hbm, out_hbm,
           src_v, dst_v, xs_v, kern_v, zero_v, acc_sh,
           sem_i, sem_k, sem_g, sem_s):
        cid = lax.axis_index("c")
        sid = lax.axis_index("s")
        wid = sid * NC + cid
        e0 = wid * epw
        s0 = ebase + e0
        d0 = E + ebase + e0

        z16 = jnp.zeros((16,), jnp.float32)
        for i in range(ZB):
            for c in range(_LANES):
                zero_v[i, pl.ds(c * 16, 16)] = z16
        for r in range(RPT // ZB):
            pltpu.sync_copy(zero_v, acc_sh.at[pl.ds(sid * RPT + r * ZB, ZB)])
        plsc.subcore_barrier()

        def issue_idx(t, ib):
            pltpu.async_copy(eidx_hbm.at[pl.ds(s0 + t * CH, CH)],
                             src_v.at[ib], sem_i.at[ib])
            pltpu.async_copy(eidx_hbm.at[pl.ds(d0 + t * CH, CH)],
                             dst_v.at[ib], sem_i.at[ib])

        def wait_idx(t, ib):
            pltpu.make_async_copy(eidx_hbm.at[pl.ds(s0 + t * CH, CH)],
                                  src_v.at[ib], sem_i.at[ib]).wait()
            pltpu.make_async_copy(eidx_hbm.at[pl.ds(d0 + t * CH, CH)],
                                  dst_v.at[ib], sem_i.at[ib]).wait()

        def issue_kern(t, b):
            pltpu.async_copy(kern_hbm.at[pl.ds(e0 + t * CH, CH)],
                             kern_v.at[b], sem_k.at[b])

        def wait_kern(t, b):
            pltpu.make_async_copy(kern_hbm.at[pl.ds(e0 + t * CH, CH)],
                                  kern_v.at[b], sem_k.at[b]).wait()

        def issue_gather(b, ib):
            pltpu.async_copy(x_hbm.at[src_v.at[ib]], xs_v.at[b], sem_g.at[b])

        def wait_gather(b, ib):
            pltpu.make_async_copy(x_hbm.at[src_v.at[ib]], xs_v.at[b],
                                  sem_g.at[b]).wait()

        def issue_scatter(b, ib):
            pltpu.async_copy(xs_v.at[b], acc_sh.at[dst_v.at[ib]],
                             sem_s.at[b], add=True)

        def wait_scatter(b, ib):
            pltpu.make_async_copy(xs_v.at[b], acc_sh.at[dst_v.at[ib]],
                                  sem_s.at[b]).wait()

        for t in (0, 1):
            issue_idx(t, t)
            issue_kern(t, t)
        wait_idx(0, 0)
        issue_gather(0, 0)

        def chunk(j, carry):
            b = lax.rem(j, 2)
            ib = lax.rem(j, 4)
            bn = lax.rem(j + 1, 2)
            ibn = lax.rem(j + 1, 4)

            @pl.when(j + 2 < nchunk)
            def _():
                issue_idx(j + 2, lax.rem(j + 2, 4))

            @pl.when(j + 1 < nchunk)
            def _():
                wait_idx(j + 1, ibn)

                @pl.when(j >= 1)
                def _():
                    wait_scatter(bn, lax.rem(j + 3, 4))

                issue_gather(bn, ibn)

            wait_kern(j, b)
            wait_gather(b, ib)

            @plsc.parallel_loop(0, CH, 1, unroll=4)
            def erow(e):
                for c in range(_LANES):
                    sl = pl.ds(c * 16, 16)
                    xs_v[b, e, sl] = xs_v[b, e, sl] * kern_v[b, e, sl]

            @pl.when(j + 2 < nchunk)
            def _():
                issue_kern(j + 2, b)

            issue_scatter(b, ib)
            return carry

        lax.fori_loop(0, nchunk, chunk, 0)
        for t in (nchunk - 2, nchunk - 1):
            wait_scatter(t % 2, t % 4)
        plsc.subcore_barrier()

        for r in range(RPT // ZB):
            off = sid * RPT + r * ZB
            pltpu.sync_copy(acc_sh.at[pl.ds(off, ZB)],
                            out_hbm.at[cid, pl.ds(off, ZB)])

    return sc(x, kern_half, eidx_flat)


def _ln_mlp(partials, x, conv_bias, ln_g, ln_b, W1T, b1, W2T, b2, layer_scale):
    nb = 2000

    def body(p_ref, x_ref, cb_ref, g_ref, b_ref, w1_ref, b1_ref,
             w2_ref, b2_ref, ls_ref, o_ref):
        x1 = p_ref[0] + p_ref[1] + cb_ref[...]
        mu = jnp.mean(x1, axis=-1, keepdims=True)
        xc = x1 - mu
        var = jnp.mean(xc * xc, axis=-1, keepdims=True)
        h = xc * lax.rsqrt(var + 1e-5) * g_ref[...] + b_ref[...]
        a = jnp.dot(h, w1_ref[...], preferred_element_type=jnp.float32)
        a = a + b1_ref[...]
        a = 0.5 * a * (1.0 + lax.erf(a * 0.7071067811865476))
        o = jnp.dot(a, w2_ref[...], preferred_element_type=jnp.float32)
        o = o + b2_ref[...]
        o_ref[...] = ls_ref[...] * o + x_ref[...]

    part_spec = pl.BlockSpec((NC, nb, C), lambda i: (0, i, 0))
    return pl.pallas_call(
        body,
        grid=(N // nb,),
        in_specs=[
            part_spec,
            pl.BlockSpec((nb, C), lambda i: (i, 0)),
            pl.BlockSpec((1, C), lambda i: (0, 0)),
            pl.BlockSpec((1, C), lambda i: (0, 0)),
            pl.BlockSpec((1, C), lambda i: (0, 0)),
            pl.BlockSpec((C, WF), lambda i: (0, 0)),
            pl.BlockSpec((1, WF), lambda i: (0, 0)),
            pl.BlockSpec((WF, C), lambda i: (0, 0)),
            pl.BlockSpec((1, C), lambda i: (0, 0)),
            pl.BlockSpec((1, C), lambda i: (0, 0)),
        ],
        out_specs=pl.BlockSpec((nb, C), lambda i: (i, 0)),
        out_shape=jax.ShapeDtypeStruct((N, C), jnp.float32),
    )(partials, x, conv_bias.reshape(1, C), ln_g.reshape(1, C),
      ln_b.reshape(1, C), W1T, b1.reshape(1, WF), W2T, b2.reshape(1, C),
      layer_scale.reshape(1, C))


def kernel(x, kernel_basis, fiber_kernel_basis, edge_index, Wk, conv_bias,
           ln_g, ln_b, W1, b1, W2, b2, layer_scale):
    eidx_flat = edge_index.reshape(2 * E)
    kern_full = _edge_kernel(kernel_basis, Wk, 0, E)
    partials = _sc_segment(x, kern_full, eidx_flat, 0, EPW, NCHUNK)
    return _ln_mlp(partials, x, conv_bias, ln_g, ln_b,
                   W1.T, b1, W2.T, b2, layer_scale)
